# baseline (speedup 1.0000x reference)
import functools

import jax
import jax.numpy as jnp
import numpy as np
from jax import lax
from jax.experimental import pallas as pl
from jax.experimental.pallas import tpu as pltpu
from jax.experimental.pallas import tpu_sc as plsc

N_NODES = 10000
SEQ = 32
EMB = 128
THID = 256
NREL = 4
N_EDGES = 320000
N_HYPER = 160000

NW = 32
NSEG = 10240
TRASH = 10200
HP = 163840
CBINS = 6 * NSEG
CBCAP = 65536
ZR = 40
EP = 327680


def _mesh():
    return plsc.VectorSubcoreMesh(core_axis_name="c", subcore_axis_name="s")


def _wid():
    return lax.axis_index("s") * 2 + lax.axis_index("c")


def _zero_vmem_rows(ref, nrows):
    z = jnp.zeros((16,), jnp.float32)

    def body(r, _):
        for q in range(EMB // 16):
            ref[r, pl.ds(q * 16, 16)] = z
        return _

    lax.fori_loop(0, nrows, body, 0)


def _make_gather(T, M, K):
    per_w = M // NW
    nch = per_w // K

    @functools.partial(
        pl.kernel,
        out_type=jax.ShapeDtypeStruct((M, EMB), jnp.float32),
        mesh=_mesh(),
        compiler_params=pltpu.CompilerParams(needs_layout_passes=False),
        scratch_types=[
            pltpu.VMEM((K,), jnp.int32),
            pltpu.VMEM((K, EMB), jnp.float32),
            pltpu.SemaphoreType.DMA,
        ],
    )
    def k(table, idx, out, idx_v, rows_v, sem):
        base = _wid() * per_w

        def body(c, carry):
            off = base + c * K
            pltpu.sync_copy(idx.at[pl.ds(off, K)], idx_v)
            pltpu.async_copy(table.at[idx_v], rows_v, sem).wait()
            pltpu.sync_copy(rows_v, out.at[pl.ds(off, K)])
            return carry

        lax.fori_loop(0, nch, body, 0)

    return k


def _make_counts():
    hyp_w = HP // NW
    edg_w = N_EDGES // NW
    KH = 640
    KE = 400

    @functools.partial(
        pl.kernel,
        out_type=jax.ShapeDtypeStruct((NW * CBCAP,), jnp.float32),
        mesh=_mesh(),
        compiler_params=pltpu.CompilerParams(needs_layout_passes=False),
        scratch_types=[
            pltpu.VMEM((CBCAP,), jnp.float32),
            pltpu.VMEM((KH,), jnp.int32),
            pltpu.VMEM((KH,), jnp.int32),
            pltpu.VMEM((KE,), jnp.int32),
            pltpu.VMEM((KE,), jnp.int32),
        ],
    )
    def k(hn, he, ed, et, out, hist, hn_v, he_v, ed_v, et_v):
        wid = _wid()
        one = jnp.ones((16,), jnp.float32)
        z16 = jnp.zeros((16,), jnp.float32)

        def zb(i, _):
            hist[pl.ds(i * 16, 16)] = z16
            return _

        lax.fori_loop(0, CBCAP // 16, zb, 0)

        def scat(iv):
            plsc.addupdate_scatter(hist, [iv], one)

        hbase = wid * hyp_w

        def hyp_body(c, _):
            off = hbase + c * KH
            pltpu.sync_copy(hn.at[pl.ds(off, KH)], hn_v)
            pltpu.sync_copy(he.at[pl.ds(off, KH)], he_v)

            def inner(q, __):
                scat(hn_v[pl.ds(q * 16, 16)])
                scat(he_v[pl.ds(q * 16, 16)] + NSEG)
                return __

            lax.fori_loop(0, KH // 16, inner, 0)
            return _

        lax.fori_loop(0, hyp_w // KH, hyp_body, 0)

        ebase = wid * edg_w

        def edg_body(c, _):
            off = ebase + c * KE
            pltpu.sync_copy(ed.at[pl.ds(off, KE)], ed_v)
            pltpu.sync_copy(et.at[pl.ds(off, KE)], et_v)

            def inner(q, __):
                scat(et_v[pl.ds(q * 16, 16)] * NSEG
                     + ed_v[pl.ds(q * 16, 16)] + 2 * NSEG)
                return __

            lax.fori_loop(0, KE // 16, inner, 0)
            return _

        lax.fori_loop(0, edg_w // KE, edg_body, 0)

        pltpu.sync_copy(hist, out.at[pl.ds(wid * CBCAP, CBCAP)])

    return k


def _counts_reduce_body(h, o):
    o[...] = jnp.sum(h[...], axis=0)


def _make_segsum(M, K):
    per_w = M // NW
    nch = per_w // K

    @functools.partial(
        pl.kernel,
        out_type=jax.ShapeDtypeStruct((2 * NSEG, EMB), jnp.float32),
        mesh=_mesh(),
        compiler_params=pltpu.CompilerParams(needs_layout_passes=False),
        scratch_types=[
            pltpu.VMEM((K,), jnp.int32),
            pltpu.VMEM((K,), jnp.int32),
            pltpu.VMEM((K, EMB), jnp.float32),
            pltpu.VMEM((ZR, EMB), jnp.float32),
            pltpu.VMEM_SHARED((NSEG, EMB), jnp.float32),
            pltpu.SemaphoreType.DMA,
        ],
    )
    def k(table, gidx, sidx, out, gidx_v, sidx_v, rows_v, zb_v, acc, sem):
        cid = lax.axis_index("c")
        sid = lax.axis_index("s")
        base = _wid() * per_w
        rows_per_tile = NSEG // 16

        _zero_vmem_rows(zb_v, ZR)
        for z in range(rows_per_tile // ZR):
            pltpu.sync_copy(zb_v, acc.at[pl.ds(sid * rows_per_tile + z * ZR, ZR)])
        plsc.subcore_barrier()

        def body(c, carry):
            off = base + c * K
            pltpu.sync_copy(gidx.at[pl.ds(off, K)], gidx_v)
            pltpu.sync_copy(sidx.at[pl.ds(off, K)], sidx_v)
            pltpu.async_copy(table.at[gidx_v], rows_v, sem).wait()
            pltpu.sync_copy(rows_v, acc.at[sidx_v], add=True)
            return carry

        lax.fori_loop(0, nch, body, 0)
        plsc.subcore_barrier()

        for z in range(rows_per_tile // ZR):
            r0 = sid * rows_per_tile + z * ZR
            pltpu.sync_copy(acc.at[pl.ds(r0, ZR)], zb_v)
            pltpu.sync_copy(zb_v, out.at[pl.ds(cid * NSEG + r0, ZR)])

    return k


def _make_prep_w(K):
    per_w = EP // NW
    nch = per_w // K

    @functools.partial(
        pl.kernel,
        out_type=(jax.ShapeDtypeStruct((EP,), jnp.int32),
                  jax.ShapeDtypeStruct((EP,), jnp.float32)),
        mesh=_mesh(),
        compiler_params=pltpu.CompilerParams(needs_layout_passes=False),
        scratch_types=[
            pltpu.VMEM((K,), jnp.int32),
            pltpu.VMEM((K,), jnp.int32),
            pltpu.VMEM((K,), jnp.int32),
            pltpu.VMEM((K,), jnp.int32),
            pltpu.VMEM((K,), jnp.float32),
            pltpu.VMEM((NREL * NSEG,), jnp.float32),
        ],
    )
    def k(U, esrc, edst, etyp, gout, wout, src_v, dst_v, typ_v, g_v, w_v, u_v):
        base = _wid() * per_w
        pltpu.sync_copy(U, u_v)

        def body(c, carry):
            off = base + c * K
            pltpu.sync_copy(esrc.at[pl.ds(off, K)], src_v)
            pltpu.sync_copy(edst.at[pl.ds(off, K)], dst_v)
            pltpu.sync_copy(etyp.at[pl.ds(off, K)], typ_v)

            def prep(q, _):
                t = typ_v[pl.ds(q * 16, 16)]
                g_v[pl.ds(q * 16, 16)] = t * N_NODES + src_v[pl.ds(q * 16, 16)]
                uix = t * NSEG + dst_v[pl.ds(q * 16, 16)]
                w_v[pl.ds(q * 16, 16)] = plsc.load_gather(u_v, [uix])
                return _

            lax.fori_loop(0, K // 16, prep, 0)
            pltpu.sync_copy(g_v, gout.at[pl.ds(off, K)])
            pltpu.sync_copy(w_v, wout.at[pl.ds(off, K)])
            return carry

        lax.fori_loop(0, nch, body, 0)

    return k


def _make_wsegsum(M, K):
    per_w = M // NW
    nch = per_w // K

    @functools.partial(
        pl.kernel,
        out_type=jax.ShapeDtypeStruct((2 * NSEG, EMB), jnp.float32),
        mesh=_mesh(),
        compiler_params=pltpu.CompilerParams(needs_layout_passes=False),
        scratch_types=[
            pltpu.VMEM((K,), jnp.int32),
            pltpu.VMEM((K,), jnp.int32),
            pltpu.VMEM((K,), jnp.float32),
            pltpu.VMEM((K, EMB), jnp.float32),
            pltpu.VMEM((ZR, EMB), jnp.float32),
            pltpu.VMEM_SHARED((NSEG, EMB), jnp.float32),
            pltpu.SemaphoreType.DMA,
        ],
    )
    def k(table, gidx, sidx, wts, out,
          gidx_v, sidx_v, w_v, rows_v, zb_v, acc, sem):
        cid = lax.axis_index("c")
        sid = lax.axis_index("s")
        base = _wid() * per_w
        rows_per_tile = NSEG // 16

        _zero_vmem_rows(zb_v, ZR)
        for z in range(rows_per_tile // ZR):
            pltpu.sync_copy(zb_v, acc.at[pl.ds(sid * rows_per_tile + z * ZR, ZR)])
        plsc.subcore_barrier()

        def body(c, carry):
            off = base + c * K
            pltpu.sync_copy(gidx.at[pl.ds(off, K)], gidx_v)
            pltpu.sync_copy(sidx.at[pl.ds(off, K)], sidx_v)
            pltpu.sync_copy(wts.at[pl.ds(off, K)], w_v)
            pltpu.async_copy(table.at[gidx_v], rows_v, sem).wait()

            def scale(e, _):
                ws = plsc.load_gather(w_v, [jnp.full((16,), e, jnp.int32)])
                for q in range(EMB // 16):
                    rows_v[e, pl.ds(q * 16, 16)] = (
                        rows_v[e, pl.ds(q * 16, 16)] * ws)
                return _

            lax.fori_loop(0, K, scale, 0)
            pltpu.sync_copy(rows_v, acc.at[sidx_v], add=True)
            return carry

        lax.fori_loop(0, nch, body, 0)
        plsc.subcore_barrier()

        for z in range(rows_per_tile // ZR):
            r0 = sid * rows_per_tile + z * ZR
            pltpu.sync_copy(acc.at[pl.ds(r0, ZR)], zb_v)
            pltpu.sync_copy(zb_v, out.at[pl.ds(cid * NSEG + r0, ZR)])

    return k


def _stage1_body(srows, seg, semb, wrow, q1w, q1b, q2w, q2b, h1w,
                 m32, m32t, eten, out):
    S = srows[...]
    wt = jnp.dot(wrow[...], m32t[...])
    Sw = S * wt.reshape(1, 1, EMB)

    s2 = jnp.sum(Sw * Sw, axis=1)
    n2 = jnp.dot(s2, m32[...])
    inv = 1.0 / jnp.maximum(jnp.sqrt(n2), 1e-12)
    invt = jnp.dot(inv, m32t[...])
    P = Sw * invt[:, None, :]

    segb = seg[...]
    r0 = semb[0, :].reshape(1, 1, EMB)
    r1 = semb[1, :].reshape(1, 1, EMB)
    r2 = semb[2, :].reshape(1, 1, EMB)
    sb = segb[:, :, None]
    G = jnp.where(sb == 0, r0, jnp.where(sb == 1, r1, r2))
    g2 = jnp.sum(G * G, axis=1)
    ng2 = jnp.dot(g2, m32[...])
    ginv = 1.0 / jnp.maximum(jnp.sqrt(ng2), 1e-12)
    ginvt = jnp.dot(ginv, m32t[...])
    P = P + G * ginvt[:, None, :]

    bn = P.shape[0]
    f = jnp.zeros((bn, EMB), jnp.float32)
    for l in range(SEQ):
        f = f + jnp.dot(P[:, l, :], eten[l])

    h = jax.nn.relu(jnp.dot(f, q1w[...]) + q1b[...])
    x = jnp.dot(h, q2w[...]) + q2b[...]
    out[...] = jnp.dot(x, h1w[...])


def _scale_body(p0, p1, b, o):
    bb = b[...]
    inv = jnp.where(bb > 0, 1.0 / jnp.maximum(bb, 1e-30), 0.0)
    o[...] = (p0[...] + p1[...]) * inv


def _combine_c_body(p0, p1, d, h1b, rel, rootw, rootb, c, yout, rout, uout):
    r = pl.program_id(1)
    dd = d[...]
    inv = jnp.where(dd > 0, 1.0 / jnp.maximum(dd, 1e-30), 0.0)
    h2x = jax.nn.relu((p0[...] + p1[...]) * inv + h1b[...])
    yout[...] = jnp.dot(h2x, rel[0])[None]

    @pl.when(r == 0)
    def _():
        rout[...] = jnp.dot(h2x, rootw[...]) + rootb[...]
        uout[...] = 1.0 / jnp.maximum(c[...], 1.0)


def _combine_d_body(root, q0, q1, h2w, o):
    x3 = jax.nn.relu(root[...] + q0[...] + q1[...])
    o[...] = jnp.dot(x3, h2w[...])


def _combine_f_body(p0, p1, d, h2b, linw, linb, o):
    dd = d[...]
    inv = jnp.where(dd > 0, 1.0 / jnp.maximum(dd, 1e-30), 0.0)
    h = jax.nn.relu((p0[...] + p1[...]) * inv + h2b[...])
    o[...] = jnp.dot(h, linw[...]) + linb[...]


def _onehot_consts():
    d = np.arange(EMB)
    m32 = np.zeros((EMB, SEQ), np.float32)
    m32[d, d % SEQ] = 1.0
    eten = np.zeros((SEQ, EMB, EMB), np.float32)
    for l in range(SEQ):
        eten[l, d, 4 * l + d // SEQ] = 1.0
    return jnp.asarray(m32), jnp.asarray(m32.T), jnp.asarray(eten)


def kernel(src, seg, edge_index, hyper_index, edge_type, src_emb, seg_emb, w,
           q1_w, q1_b, q2_w, q2_b, h1_w, h1_b, rg_rel, rg_root, rg_b,
           h2_w, h2_b, lin_w, lin_b):
    m32, m32t, eten = _onehot_consts()
    f32 = jnp.float32

    srows = _make_gather(src_emb.shape[0], N_NODES * SEQ, 400)(
        src_emb, src.reshape(-1))

    hn = hyper_index[0]
    he = hyper_index[1]
    pad = HP - N_HYPER
    trash_pad = N_NODES + (jnp.arange(pad, dtype=jnp.int32) % (NSEG - N_NODES))
    zero_pad = jnp.zeros((pad,), jnp.int32)
    hn_t = jnp.concatenate([hn, trash_pad])
    he_t = jnp.concatenate([he, trash_pad])
    hn_z = jnp.concatenate([hn, zero_pad])
    he_z = jnp.concatenate([he, zero_pad])
    counts_raw = _make_counts()(hn_t, he_t, edge_index[1], edge_type)
    cnt = pl.pallas_call(
        _counts_reduce_body,
        grid=(16,),
        in_specs=[pl.BlockSpec((NW, 32, EMB), lambda i: (0, i, 0))],
        out_specs=pl.BlockSpec((32, EMB), lambda i: (i, 0)),
        out_shape=jax.ShapeDtypeStruct((CBCAP // EMB, EMB), f32),
    )(counts_raw.reshape(NW, CBCAP // EMB, EMB)).reshape(CBCAP)
    d_col = cnt[:N_NODES].reshape(N_NODES, 1)
    b_col = cnt[NSEG:NSEG + N_NODES].reshape(N_NODES, 1)
    c_rg = cnt[2 * NSEG:6 * NSEG].reshape(1, NREL * NSEG)

    bn = 200
    nb = N_NODES // bn
    full = lambda shape: pl.BlockSpec(shape, lambda i: tuple(0 for _ in shape))
    xl1 = pl.pallas_call(
        _stage1_body,
        grid=(nb,),
        in_specs=[
            pl.BlockSpec((bn, SEQ, EMB), lambda i: (i, 0, 0)),
            pl.BlockSpec((bn, SEQ), lambda i: (i, 0)),
            full((8, EMB)),
            full((1, SEQ)),
            full((EMB, THID)),
            full((1, THID)),
            full((THID, EMB)),
            full((1, EMB)),
            full((EMB, EMB)),
            full((EMB, SEQ)),
            full((SEQ, EMB)),
            full((SEQ, EMB, EMB)),
        ],
        out_specs=pl.BlockSpec((bn, EMB), lambda i: (i, 0)),
        out_shape=jax.ShapeDtypeStruct((N_NODES, EMB), f32),
    )(srows.reshape(N_NODES, SEQ, EMB), seg,
      jnp.concatenate([seg_emb, jnp.zeros((5, EMB), f32)], axis=0),
      w.reshape(1, SEQ), q1_w, q1_b.reshape(1, THID), q2_w,
      q2_b.reshape(1, EMB), h1_w, m32, m32t, eten)

    seg_hyp = _make_segsum(HP, 160)
    bs = lambda: pl.BlockSpec((1000, EMB), lambda i: (i, 0))
    cs = lambda: pl.BlockSpec((1000, 1), lambda i: (i, 0))
    scale_call = lambda p, col: pl.pallas_call(
        _scale_body,
        grid=(10,),
        in_specs=[bs(), bs(), cs()],
        out_specs=bs(),
        out_shape=jax.ShapeDtypeStruct((N_NODES, EMB), f32),
    )(p[0, :N_NODES], p[1, :N_NODES], col)

    p1h = seg_hyp(xl1, hn_z, he_t).reshape(2, NSEG, EMB)
    e1 = scale_call(p1h, b_col)
    p2h = seg_hyp(e1, he_z, hn_t).reshape(2, NSEG, EMB)

    ub = NREL * NSEG // 10
    yout, root, u = pl.pallas_call(
        _combine_c_body,
        grid=(10, NREL),
        in_specs=[
            pl.BlockSpec((1000, EMB), lambda i, r: (i, 0)),
            pl.BlockSpec((1000, EMB), lambda i, r: (i, 0)),
            pl.BlockSpec((1000, 1), lambda i, r: (i, 0)),
            pl.BlockSpec((1, EMB), lambda i, r: (0, 0)),
            pl.BlockSpec((1, EMB, EMB), lambda i, r: (r, 0, 0)),
            pl.BlockSpec((EMB, EMB), lambda i, r: (0, 0)),
            pl.BlockSpec((1, EMB), lambda i, r: (0, 0)),
            pl.BlockSpec((1, ub), lambda i, r: (0, i)),
        ],
        out_specs=[
            pl.BlockSpec((1, 1000, EMB), lambda i, r: (r, i, 0)),
            pl.BlockSpec((1000, EMB), lambda i, r: (i, 0)),
            pl.BlockSpec((1, ub), lambda i, r: (0, i)),
        ],
        out_shape=[
            jax.ShapeDtypeStruct((NREL, N_NODES, EMB), f32),
            jax.ShapeDtypeStruct((N_NODES, EMB), f32),
            jax.ShapeDtypeStruct((1, NREL * NSEG), f32),
        ],
    )(p2h[0, :N_NODES], p2h[1, :N_NODES], d_col,
      h1_b.reshape(1, EMB), rg_rel, rg_root, rg_b.reshape(1, EMB), c_rg)

    epad = EP - N_EDGES
    ez = jnp.zeros((epad,), jnp.int32)
    esrc_p = jnp.concatenate([edge_index[0], ez])
    edst_p = jnp.concatenate(
        [edge_index[1],
         N_NODES + (jnp.arange(epad, dtype=jnp.int32) % (NSEG - N_NODES))])
    etyp_p = jnp.concatenate([edge_type, ez])
    egidx, ew = _make_prep_w(160)(u.reshape(NREL * NSEG), esrc_p, edst_p, etyp_p)
    qp = _make_wsegsum(EP, 160)(
        yout.reshape(NREL * N_NODES, EMB), egidx, edst_p, ew
    ).reshape(2, NSEG, EMB)

    xl2 = pl.pallas_call(
        _combine_d_body,
        grid=(10,),
        in_specs=[bs(), bs(), bs(), full((EMB, EMB))],
        out_specs=bs(),
        out_shape=jax.ShapeDtypeStruct((N_NODES, EMB), f32),
    )(root, qp[0, :N_NODES], qp[1, :N_NODES], h2_w)

    p3h = seg_hyp(xl2, hn_z, he_t).reshape(2, NSEG, EMB)
    e2 = scale_call(p3h, b_col)
    p4h = seg_hyp(e2, he_z, hn_t).reshape(2, NSEG, EMB)

    out = pl.pallas_call(
        _combine_f_body,
        grid=(10,),
        in_specs=[bs(), bs(), cs(), full((1, EMB)),
                  full((EMB, EMB)), full((1, EMB))],
        out_specs=bs(),
        out_shape=jax.ShapeDtypeStruct((N_NODES, EMB), f32),
    )(p4h[0, :N_NODES], p4h[1, :N_NODES], d_col,
      h2_b.reshape(1, EMB), lin_w, lin_b.reshape(1, EMB))
    return out


# speedup vs baseline: 1.1272x; 1.1272x over previous
import functools

import jax
import jax.numpy as jnp
import numpy as np
from jax import lax
from jax.experimental import pallas as pl
from jax.experimental.pallas import tpu as pltpu
from jax.experimental.pallas import tpu_sc as plsc

N_NODES = 10000
SEQ = 32
EMB = 128
THID = 256
NREL = 4
N_EDGES = 320000
N_HYPER = 160000

NW = 32
NSEG = 10240
TRASH = 10200
HP = 163840
CBINS = 6 * NSEG
CBCAP = 65536
ZR = 40
EP = 327680


def _mesh():
    return plsc.VectorSubcoreMesh(core_axis_name="c", subcore_axis_name="s")


def _wid():
    return lax.axis_index("s") * 2 + lax.axis_index("c")


def _zero_vmem_rows(ref, nrows):
    z = jnp.zeros((16,), jnp.float32)

    def body(r, _):
        for q in range(EMB // 16):
            ref[r, pl.ds(q * 16, 16)] = z
        return _

    lax.fori_loop(0, nrows, body, 0)


def _make_gather(T, M, K):
    per_w = M // NW
    nch = per_w // K

    @functools.partial(
        pl.kernel,
        out_type=jax.ShapeDtypeStruct((M, EMB), jnp.float32),
        mesh=_mesh(),
        compiler_params=pltpu.CompilerParams(needs_layout_passes=False),
        scratch_types=[
            pltpu.VMEM((K,), jnp.int32),
            pltpu.VMEM((K, EMB), jnp.float32),
            pltpu.SemaphoreType.DMA,
        ],
    )
    def k(table, idx, out, idx_v, rows_v, sem):
        base = _wid() * per_w

        def body(c, carry):
            off = base + c * K
            pltpu.sync_copy(idx.at[pl.ds(off, K)], idx_v)
            pltpu.async_copy(table.at[idx_v], rows_v, sem).wait()
            pltpu.sync_copy(rows_v, out.at[pl.ds(off, K)])
            return carry

        lax.fori_loop(0, nch, body, 0)

    return k


def _make_counts():
    hyp_w = HP // NW
    edg_w = N_EDGES // NW
    KH = 640
    KE = 400

    @functools.partial(
        pl.kernel,
        out_type=jax.ShapeDtypeStruct((NW * CBCAP,), jnp.float32),
        mesh=_mesh(),
        compiler_params=pltpu.CompilerParams(needs_layout_passes=False),
        scratch_types=[
            pltpu.VMEM((CBCAP,), jnp.float32),
            pltpu.VMEM((KH,), jnp.int32),
            pltpu.VMEM((KH,), jnp.int32),
            pltpu.VMEM((KE,), jnp.int32),
            pltpu.VMEM((KE,), jnp.int32),
        ],
    )
    def k(hn, he, ed, et, out, hist, hn_v, he_v, ed_v, et_v):
        wid = _wid()
        one = jnp.ones((16,), jnp.float32)
        z16 = jnp.zeros((16,), jnp.float32)

        def zb(i, _):
            hist[pl.ds(i * 16, 16)] = z16
            return _

        lax.fori_loop(0, CBCAP // 16, zb, 0)

        def scat(iv):
            plsc.addupdate_scatter(hist, [iv], one)

        hbase = wid * hyp_w

        def hyp_body(c, _):
            off = hbase + c * KH
            pltpu.sync_copy(hn.at[pl.ds(off, KH)], hn_v)
            pltpu.sync_copy(he.at[pl.ds(off, KH)], he_v)

            def inner(q, __):
                scat(hn_v[pl.ds(q * 16, 16)])
                scat(he_v[pl.ds(q * 16, 16)] + NSEG)
                return __

            lax.fori_loop(0, KH // 16, inner, 0)
            return _

        lax.fori_loop(0, hyp_w // KH, hyp_body, 0)

        ebase = wid * edg_w

        def edg_body(c, _):
            off = ebase + c * KE
            pltpu.sync_copy(ed.at[pl.ds(off, KE)], ed_v)
            pltpu.sync_copy(et.at[pl.ds(off, KE)], et_v)

            def inner(q, __):
                scat(et_v[pl.ds(q * 16, 16)] * NSEG
                     + ed_v[pl.ds(q * 16, 16)] + 2 * NSEG)
                return __

            lax.fori_loop(0, KE // 16, inner, 0)
            return _

        lax.fori_loop(0, edg_w // KE, edg_body, 0)

        pltpu.sync_copy(hist, out.at[pl.ds(wid * CBCAP, CBCAP)])

    return k


def _counts_reduce_body(h, o):
    o[...] = jnp.sum(h[...], axis=0)


def _make_segsum(M, K, na, nb):
    assert 16 * (na + nb) * K == M

    @functools.partial(
        pl.kernel,
        out_type=jax.ShapeDtypeStruct((2 * NSEG, EMB), jnp.float32),
        mesh=_mesh(),
        compiler_params=pltpu.CompilerParams(needs_layout_passes=False),
        scratch_types=[
            pltpu.VMEM((K,), jnp.int32),
            pltpu.VMEM((K,), jnp.int32),
            pltpu.VMEM((K, EMB), jnp.float32),
            pltpu.VMEM((ZR, EMB), jnp.float32),
            pltpu.VMEM_SHARED((NSEG, EMB), jnp.float32),
            pltpu.SemaphoreType.DMA,
        ],
    )
    def k(table, gidx, sidx, out, gidx_v, sidx_v, rows_v, zb_v, acc, sem):
        cid = lax.axis_index("c")
        sid = lax.axis_index("s")
        base = jnp.where(cid == 0, sid * (na * K), 16 * na * K + sid * (nb * K))
        nch = jnp.where(cid == 0, na, nb)
        rows_per_tile = NSEG // 16

        _zero_vmem_rows(zb_v, ZR)
        for z in range(rows_per_tile // ZR):
            pltpu.sync_copy(zb_v, acc.at[pl.ds(sid * rows_per_tile + z * ZR, ZR)])
        plsc.subcore_barrier()

        def body(c, carry):
            off = base + c * K
            pltpu.sync_copy(gidx.at[pl.ds(off, K)], gidx_v)
            pltpu.sync_copy(sidx.at[pl.ds(off, K)], sidx_v)
            pltpu.async_copy(table.at[gidx_v], rows_v, sem).wait()
            pltpu.sync_copy(rows_v, acc.at[sidx_v], add=True)
            return carry

        lax.fori_loop(0, nch, body, 0)
        plsc.subcore_barrier()

        for z in range(rows_per_tile // ZR):
            r0 = sid * rows_per_tile + z * ZR
            pltpu.sync_copy(acc.at[pl.ds(r0, ZR)], zb_v)
            pltpu.sync_copy(zb_v, out.at[pl.ds(cid * NSEG + r0, ZR)])

    return k


def _make_prep_w(K):
    per_w = EP // NW
    nch = per_w // K

    @functools.partial(
        pl.kernel,
        out_type=(jax.ShapeDtypeStruct((EP,), jnp.int32),
                  jax.ShapeDtypeStruct((EP,), jnp.float32)),
        mesh=_mesh(),
        compiler_params=pltpu.CompilerParams(needs_layout_passes=False),
        scratch_types=[
            pltpu.VMEM((K,), jnp.int32),
            pltpu.VMEM((K,), jnp.int32),
            pltpu.VMEM((K,), jnp.int32),
            pltpu.VMEM((K,), jnp.int32),
            pltpu.VMEM((K,), jnp.float32),
            pltpu.VMEM((NREL * NSEG,), jnp.float32),
        ],
    )
    def k(U, esrc, edst, etyp, gout, wout, src_v, dst_v, typ_v, g_v, w_v, u_v):
        base = _wid() * per_w
        pltpu.sync_copy(U, u_v)

        def body(c, carry):
            off = base + c * K
            pltpu.sync_copy(esrc.at[pl.ds(off, K)], src_v)
            pltpu.sync_copy(edst.at[pl.ds(off, K)], dst_v)
            pltpu.sync_copy(etyp.at[pl.ds(off, K)], typ_v)

            def prep(q, _):
                t = typ_v[pl.ds(q * 16, 16)]
                g_v[pl.ds(q * 16, 16)] = t * N_NODES + src_v[pl.ds(q * 16, 16)]
                uix = t * NSEG + dst_v[pl.ds(q * 16, 16)]
                w_v[pl.ds(q * 16, 16)] = plsc.load_gather(u_v, [uix])
                return _

            lax.fori_loop(0, K // 16, prep, 0)
            pltpu.sync_copy(g_v, gout.at[pl.ds(off, K)])
            pltpu.sync_copy(w_v, wout.at[pl.ds(off, K)])
            return carry

        lax.fori_loop(0, nch, body, 0)

    return k


def _make_wsegsum(M, K, na, nb):
    assert 16 * (na + nb) * K == M

    @functools.partial(
        pl.kernel,
        out_type=jax.ShapeDtypeStruct((2 * NSEG, EMB), jnp.float32),
        mesh=_mesh(),
        compiler_params=pltpu.CompilerParams(needs_layout_passes=False),
        scratch_types=[
            pltpu.VMEM((K,), jnp.int32),
            pltpu.VMEM((K,), jnp.int32),
            pltpu.VMEM((K,), jnp.float32),
            pltpu.VMEM((K, EMB), jnp.float32),
            pltpu.VMEM((ZR, EMB), jnp.float32),
            pltpu.VMEM_SHARED((NSEG, EMB), jnp.float32),
            pltpu.SemaphoreType.DMA,
        ],
    )
    def k(table, gidx, sidx, wts, out,
          gidx_v, sidx_v, w_v, rows_v, zb_v, acc, sem):
        cid = lax.axis_index("c")
        sid = lax.axis_index("s")
        base = jnp.where(cid == 0, sid * (na * K), 16 * na * K + sid * (nb * K))
        nch = jnp.where(cid == 0, na, nb)
        rows_per_tile = NSEG // 16

        _zero_vmem_rows(zb_v, ZR)
        for z in range(rows_per_tile // ZR):
            pltpu.sync_copy(zb_v, acc.at[pl.ds(sid * rows_per_tile + z * ZR, ZR)])
        plsc.subcore_barrier()

        def body(c, carry):
            off = base + c * K
            pltpu.sync_copy(gidx.at[pl.ds(off, K)], gidx_v)
            pltpu.sync_copy(sidx.at[pl.ds(off, K)], sidx_v)
            pltpu.sync_copy(wts.at[pl.ds(off, K)], w_v)
            pltpu.async_copy(table.at[gidx_v], rows_v, sem).wait()

            def scale(e, _):
                ws = plsc.load_gather(w_v, [jnp.full((16,), e, jnp.int32)])
                for q in range(EMB // 16):
                    rows_v[e, pl.ds(q * 16, 16)] = (
                        rows_v[e, pl.ds(q * 16, 16)] * ws)
                return _

            lax.fori_loop(0, K, scale, 0)
            pltpu.sync_copy(rows_v, acc.at[sidx_v], add=True)
            return carry

        lax.fori_loop(0, nch, body, 0)
        plsc.subcore_barrier()

        for z in range(rows_per_tile // ZR):
            r0 = sid * rows_per_tile + z * ZR
            pltpu.sync_copy(acc.at[pl.ds(r0, ZR)], zb_v)
            pltpu.sync_copy(zb_v, out.at[pl.ds(cid * NSEG + r0, ZR)])

    return k


def _stage1_body(srows, seg, semb, wrow, q1w, q1b, q2w, q2b, h1w,
                 m32, m32t, eten, out):
    S = srows[...]
    wt = jnp.dot(wrow[...], m32t[...])
    Sw = S * wt.reshape(1, 1, EMB)

    s2 = jnp.sum(Sw * Sw, axis=1)
    n2 = jnp.dot(s2, m32[...])
    inv = 1.0 / jnp.maximum(jnp.sqrt(n2), 1e-12)
    invt = jnp.dot(inv, m32t[...])
    P = Sw * invt[:, None, :]

    segb = seg[...]
    r0 = semb[0, :].reshape(1, 1, EMB)
    r1 = semb[1, :].reshape(1, 1, EMB)
    r2 = semb[2, :].reshape(1, 1, EMB)
    sb = segb[:, :, None]
    G = jnp.where(sb == 0, r0, jnp.where(sb == 1, r1, r2))
    g2 = jnp.sum(G * G, axis=1)
    ng2 = jnp.dot(g2, m32[...])
    ginv = 1.0 / jnp.maximum(jnp.sqrt(ng2), 1e-12)
    ginvt = jnp.dot(ginv, m32t[...])
    P = P + G * ginvt[:, None, :]

    bn = P.shape[0]
    f = jnp.zeros((bn, EMB), jnp.float32)
    for l in range(SEQ):
        f = f + jnp.dot(P[:, l, :], eten[l])

    h = jax.nn.relu(jnp.dot(f, q1w[...]) + q1b[...])
    x = jnp.dot(h, q2w[...]) + q2b[...]
    out[...] = jnp.dot(x, h1w[...])


def _scale_body(p0, p1, b, o):
    bb = b[...]
    inv = jnp.where(bb > 0, 1.0 / jnp.maximum(bb, 1e-30), 0.0)
    o[...] = (p0[...] + p1[...]) * inv


def _combine_c_body(p0, p1, d, h1b, rel, rootw, rootb, c, yout, rout, uout):
    r = pl.program_id(1)
    dd = d[...]
    inv = jnp.where(dd > 0, 1.0 / jnp.maximum(dd, 1e-30), 0.0)
    h2x = jax.nn.relu((p0[...] + p1[...]) * inv + h1b[...])
    yout[...] = jnp.dot(h2x, rel[0])[None]

    @pl.when(r == 0)
    def _():
        rout[...] = jnp.dot(h2x, rootw[...]) + rootb[...]
        uout[...] = 1.0 / jnp.maximum(c[...], 1.0)


def _combine_d_body(root, q0, q1, h2w, o):
    x3 = jax.nn.relu(root[...] + q0[...] + q1[...])
    o[...] = jnp.dot(x3, h2w[...])


def _combine_f_body(p0, p1, d, h2b, linw, linb, o):
    dd = d[...]
    inv = jnp.where(dd > 0, 1.0 / jnp.maximum(dd, 1e-30), 0.0)
    h = jax.nn.relu((p0[...] + p1[...]) * inv + h2b[...])
    o[...] = jnp.dot(h, linw[...]) + linb[...]


def _onehot_consts():
    d = np.arange(EMB)
    m32 = np.zeros((EMB, SEQ), np.float32)
    m32[d, d % SEQ] = 1.0
    eten = np.zeros((SEQ, EMB, EMB), np.float32)
    for l in range(SEQ):
        eten[l, d, 4 * l + d // SEQ] = 1.0
    return jnp.asarray(m32), jnp.asarray(m32.T), jnp.asarray(eten)


def kernel(src, seg, edge_index, hyper_index, edge_type, src_emb, seg_emb, w,
           q1_w, q1_b, q2_w, q2_b, h1_w, h1_b, rg_rel, rg_root, rg_b,
           h2_w, h2_b, lin_w, lin_b):
    m32, m32t, eten = _onehot_consts()
    f32 = jnp.float32

    srows = _make_gather(src_emb.shape[0], N_NODES * SEQ, 400)(
        src_emb, src.reshape(-1))

    hn = hyper_index[0]
    he = hyper_index[1]
    pad = HP - N_HYPER
    trash_pad = N_NODES + (jnp.arange(pad, dtype=jnp.int32) % (NSEG - N_NODES))
    zero_pad = jnp.zeros((pad,), jnp.int32)
    hn_t = jnp.concatenate([hn, trash_pad])
    he_t = jnp.concatenate([he, trash_pad])
    hn_z = jnp.concatenate([hn, zero_pad])
    he_z = jnp.concatenate([he, zero_pad])
    counts_raw = _make_counts()(hn_t, he_t, edge_index[1], edge_type)
    cnt = pl.pallas_call(
        _counts_reduce_body,
        grid=(16,),
        in_specs=[pl.BlockSpec((NW, 32, EMB), lambda i: (0, i, 0))],
        out_specs=pl.BlockSpec((32, EMB), lambda i: (i, 0)),
        out_shape=jax.ShapeDtypeStruct((CBCAP // EMB, EMB), f32),
    )(counts_raw.reshape(NW, CBCAP // EMB, EMB)).reshape(CBCAP)
    d_col = cnt[:N_NODES].reshape(N_NODES, 1)
    b_col = cnt[NSEG:NSEG + N_NODES].reshape(N_NODES, 1)
    c_rg = cnt[2 * NSEG:6 * NSEG].reshape(1, NREL * NSEG)

    bn = 200
    nb = N_NODES // bn
    full = lambda shape: pl.BlockSpec(shape, lambda i: tuple(0 for _ in shape))
    xl1 = pl.pallas_call(
        _stage1_body,
        grid=(nb,),
        in_specs=[
            pl.BlockSpec((bn, SEQ, EMB), lambda i: (i, 0, 0)),
            pl.BlockSpec((bn, SEQ), lambda i: (i, 0)),
            full((8, EMB)),
            full((1, SEQ)),
            full((EMB, THID)),
            full((1, THID)),
            full((THID, EMB)),
            full((1, EMB)),
            full((EMB, EMB)),
            full((EMB, SEQ)),
            full((SEQ, EMB)),
            full((SEQ, EMB, EMB)),
        ],
        out_specs=pl.BlockSpec((bn, EMB), lambda i: (i, 0)),
        out_shape=jax.ShapeDtypeStruct((N_NODES, EMB), f32),
    )(srows.reshape(N_NODES, SEQ, EMB), seg,
      jnp.concatenate([seg_emb, jnp.zeros((5, EMB), f32)], axis=0),
      w.reshape(1, SEQ), q1_w, q1_b.reshape(1, THID), q2_w,
      q2_b.reshape(1, EMB), h1_w, m32, m32t, eten)

    seg_hyp = _make_segsum(HP, 160, 44, 20)
    bs = lambda: pl.BlockSpec((1000, EMB), lambda i: (i, 0))
    cs = lambda: pl.BlockSpec((1000, 1), lambda i: (i, 0))
    scale_call = lambda p, col: pl.pallas_call(
        _scale_body,
        grid=(10,),
        in_specs=[bs(), bs(), cs()],
        out_specs=bs(),
        out_shape=jax.ShapeDtypeStruct((N_NODES, EMB), f32),
    )(p[0, :N_NODES], p[1, :N_NODES], col)

    p1h = seg_hyp(xl1, hn_z, he_t).reshape(2, NSEG, EMB)
    e1 = scale_call(p1h, b_col)
    p2h = seg_hyp(e1, he_z, hn_t).reshape(2, NSEG, EMB)

    ub = NREL * NSEG // 10
    yout, root, u = pl.pallas_call(
        _combine_c_body,
        grid=(10, NREL),
        in_specs=[
            pl.BlockSpec((1000, EMB), lambda i, r: (i, 0)),
            pl.BlockSpec((1000, EMB), lambda i, r: (i, 0)),
            pl.BlockSpec((1000, 1), lambda i, r: (i, 0)),
            pl.BlockSpec((1, EMB), lambda i, r: (0, 0)),
            pl.BlockSpec((1, EMB, EMB), lambda i, r: (r, 0, 0)),
            pl.BlockSpec((EMB, EMB), lambda i, r: (0, 0)),
            pl.BlockSpec((1, EMB), lambda i, r: (0, 0)),
            pl.BlockSpec((1, ub), lambda i, r: (0, i)),
        ],
        out_specs=[
            pl.BlockSpec((1, 1000, EMB), lambda i, r: (r, i, 0)),
            pl.BlockSpec((1000, EMB), lambda i, r: (i, 0)),
            pl.BlockSpec((1, ub), lambda i, r: (0, i)),
        ],
        out_shape=[
            jax.ShapeDtypeStruct((NREL, N_NODES, EMB), f32),
            jax.ShapeDtypeStruct((N_NODES, EMB), f32),
            jax.ShapeDtypeStruct((1, NREL * NSEG), f32),
        ],
    )(p2h[0, :N_NODES], p2h[1, :N_NODES], d_col,
      h1_b.reshape(1, EMB), rg_rel, rg_root, rg_b.reshape(1, EMB), c_rg)

    epad = EP - N_EDGES
    ez = jnp.zeros((epad,), jnp.int32)
    esrc_p = jnp.concatenate([edge_index[0], ez])
    edst_p = jnp.concatenate(
        [edge_index[1],
         N_NODES + (jnp.arange(epad, dtype=jnp.int32) % (NSEG - N_NODES))])
    etyp_p = jnp.concatenate([edge_type, ez])
    egidx, ew = _make_prep_w(160)(u.reshape(NREL * NSEG), esrc_p, edst_p, etyp_p)
    qp = _make_wsegsum(EP, 160, 86, 42)(
        yout.reshape(NREL * N_NODES, EMB), egidx, edst_p, ew
    ).reshape(2, NSEG, EMB)

    xl2 = pl.pallas_call(
        _combine_d_body,
        grid=(10,),
        in_specs=[bs(), bs(), bs(), full((EMB, EMB))],
        out_specs=bs(),
        out_shape=jax.ShapeDtypeStruct((N_NODES, EMB), f32),
    )(root, qp[0, :N_NODES], qp[1, :N_NODES], h2_w)

    p3h = seg_hyp(xl2, hn_z, he_t).reshape(2, NSEG, EMB)
    e2 = scale_call(p3h, b_col)
    p4h = seg_hyp(e2, he_z, hn_t).reshape(2, NSEG, EMB)

    out = pl.pallas_call(
        _combine_f_body,
        grid=(10,),
        in_specs=[bs(), bs(), cs(), full((1, EMB)),
                  full((EMB, EMB)), full((1, EMB))],
        out_specs=bs(),
        out_shape=jax.ShapeDtypeStruct((N_NODES, EMB), f32),
    )(p4h[0, :N_NODES], p4h[1, :N_NODES], d_col,
      h2_b.reshape(1, EMB), lin_w, lin_b.reshape(1, EMB))
    return out


# speedup vs baseline: 1.3491x; 1.1968x over previous
import functools

import jax
import jax.numpy as jnp
import numpy as np
from jax import lax
from jax.experimental import pallas as pl
from jax.experimental.pallas import tpu as pltpu
from jax.experimental.pallas import tpu_sc as plsc

N_NODES = 10000
SEQ = 32
EMB = 128
THID = 256
NREL = 4
N_EDGES = 320000
N_HYPER = 160000

NW = 32
NSEG = 10240
TRASH = 10200
HP = 163840
CBINS = 6 * NSEG
CBCAP = 65536
ZR = 40
EP = 327680


def _mesh():
    return plsc.VectorSubcoreMesh(core_axis_name="c", subcore_axis_name="s")


def _wid():
    return lax.axis_index("s") * 2 + lax.axis_index("c")


def _zero_vmem_rows(ref, nrows):
    z = jnp.zeros((16,), jnp.float32)

    def body(r, _):
        for q in range(EMB // 16):
            ref[r, pl.ds(q * 16, 16)] = z
        return _

    lax.fori_loop(0, nrows, body, 0)


def _make_gather(T, M, K):
    per_w = M // NW
    nch = per_w // K

    @functools.partial(
        pl.kernel,
        out_type=jax.ShapeDtypeStruct((M, EMB), jnp.float32),
        mesh=_mesh(),
        compiler_params=pltpu.CompilerParams(needs_layout_passes=False),
        scratch_types=[
            pltpu.VMEM((K,), jnp.int32),
            pltpu.VMEM((K,), jnp.int32),
            pltpu.VMEM((K, EMB), jnp.float32),
            pltpu.VMEM((K, EMB), jnp.float32),
            pltpu.SemaphoreType.DMA,
            pltpu.SemaphoreType.DMA,
        ],
    )
    def k(table, idx, out, idx_a, idx_b, rows_a, rows_b, sem_a, sem_b):
        base = _wid() * per_w

        def stage(c, idx_v):
            pltpu.sync_copy(idx.at[pl.ds(base + c * K, K)], idx_v)

        stage(0, idx_a)
        pltpu.async_copy(table.at[idx_a], rows_a, sem_a)

        def body(i, carry):
            c = 2 * i
            stage(c + 1, idx_b)
            pltpu.async_copy(table.at[idx_b], rows_b, sem_b)
            pltpu.make_async_copy(table.at[idx_a], rows_a, sem_a).wait()
            pltpu.sync_copy(rows_a, out.at[pl.ds(base + c * K, K)])

            @pl.when(c + 2 < nch)
            def _():
                stage(c + 2, idx_a)
                pltpu.async_copy(table.at[idx_a], rows_a, sem_a)

            pltpu.make_async_copy(table.at[idx_b], rows_b, sem_b).wait()
            pltpu.sync_copy(rows_b, out.at[pl.ds(base + (c + 1) * K, K)])
            return carry

        lax.fori_loop(0, nch // 2, body, 0)

    return k


def _make_counts():
    hyp_w = HP // NW
    edg_w = N_EDGES // NW
    KH = 640
    KE = 400

    @functools.partial(
        pl.kernel,
        out_type=jax.ShapeDtypeStruct((NW * CBCAP,), jnp.float32),
        mesh=_mesh(),
        compiler_params=pltpu.CompilerParams(needs_layout_passes=False),
        scratch_types=[
            pltpu.VMEM((CBCAP,), jnp.float32),
            pltpu.VMEM((KH,), jnp.int32),
            pltpu.VMEM((KH,), jnp.int32),
            pltpu.VMEM((KE,), jnp.int32),
            pltpu.VMEM((KE,), jnp.int32),
        ],
    )
    def k(hn, he, ed, et, out, hist, hn_v, he_v, ed_v, et_v):
        wid = _wid()
        one = jnp.ones((16,), jnp.float32)
        z16 = jnp.zeros((16,), jnp.float32)

        def zb(i, _):
            hist[pl.ds(i * 16, 16)] = z16
            return _

        lax.fori_loop(0, CBCAP // 16, zb, 0)

        def scat(iv):
            plsc.addupdate_scatter(hist, [iv], one)

        hbase = wid * hyp_w

        def hyp_body(c, _):
            off = hbase + c * KH
            pltpu.sync_copy(hn.at[pl.ds(off, KH)], hn_v)
            pltpu.sync_copy(he.at[pl.ds(off, KH)], he_v)

            def inner(q, __):
                scat(hn_v[pl.ds(q * 16, 16)])
                scat(he_v[pl.ds(q * 16, 16)] + NSEG)
                return __

            lax.fori_loop(0, KH // 16, inner, 0)
            return _

        lax.fori_loop(0, hyp_w // KH, hyp_body, 0)

        ebase = wid * edg_w

        def edg_body(c, _):
            off = ebase + c * KE
            pltpu.sync_copy(ed.at[pl.ds(off, KE)], ed_v)
            pltpu.sync_copy(et.at[pl.ds(off, KE)], et_v)

            def inner(q, __):
                scat(et_v[pl.ds(q * 16, 16)] * NSEG
                     + ed_v[pl.ds(q * 16, 16)] + 2 * NSEG)
                return __

            lax.fori_loop(0, KE // 16, inner, 0)
            return _

        lax.fori_loop(0, edg_w // KE, edg_body, 0)

        pltpu.sync_copy(hist, out.at[pl.ds(wid * CBCAP, CBCAP)])

    return k


def _counts_reduce_body(h, o):
    o[...] = jnp.sum(h[...], axis=0)


def _make_segsum(M, K, na, nb):
    assert 16 * (na + nb) * K == M

    @functools.partial(
        pl.kernel,
        out_type=jax.ShapeDtypeStruct((2 * NSEG, EMB), jnp.float32),
        mesh=_mesh(),
        compiler_params=pltpu.CompilerParams(needs_layout_passes=False),
        scratch_types=[
            pltpu.VMEM((K,), jnp.int32),
            pltpu.VMEM((K,), jnp.int32),
            pltpu.VMEM((K,), jnp.int32),
            pltpu.VMEM((K,), jnp.int32),
            pltpu.VMEM((K, EMB), jnp.float32),
            pltpu.VMEM((K, EMB), jnp.float32),
            pltpu.VMEM((ZR, EMB), jnp.float32),
            pltpu.VMEM_SHARED((NSEG, EMB), jnp.float32),
            pltpu.SemaphoreType.DMA,
            pltpu.SemaphoreType.DMA,
        ],
    )
    def k(table, gidx, sidx, out, gidx_a, gidx_b, sidx_a, sidx_b,
          rows_a, rows_b, zb_v, acc, sem_a, sem_b):
        cid = lax.axis_index("c")
        sid = lax.axis_index("s")
        base = jnp.where(cid == 0, sid * (na * K), 16 * na * K + sid * (nb * K))
        nch = jnp.where(cid == 0, na, nb)
        rows_per_tile = NSEG // 16

        _zero_vmem_rows(zb_v, ZR)
        for z in range(rows_per_tile // ZR):
            pltpu.sync_copy(zb_v, acc.at[pl.ds(sid * rows_per_tile + z * ZR, ZR)])
        plsc.subcore_barrier()

        def stage(c, g_v, s_v):
            off = base + c * K
            pltpu.sync_copy(gidx.at[pl.ds(off, K)], g_v)
            pltpu.sync_copy(sidx.at[pl.ds(off, K)], s_v)

        stage(0, gidx_a, sidx_a)
        pltpu.async_copy(table.at[gidx_a], rows_a, sem_a)

        def body(i, carry):
            c = 2 * i
            stage(c + 1, gidx_b, sidx_b)
            pltpu.async_copy(table.at[gidx_b], rows_b, sem_b)
            pltpu.make_async_copy(table.at[gidx_a], rows_a, sem_a).wait()
            pltpu.sync_copy(rows_a, acc.at[sidx_a], add=True)

            @pl.when(c + 2 < nch)
            def _():
                stage(c + 2, gidx_a, sidx_a)
                pltpu.async_copy(table.at[gidx_a], rows_a, sem_a)

            pltpu.make_async_copy(table.at[gidx_b], rows_b, sem_b).wait()
            pltpu.sync_copy(rows_b, acc.at[sidx_b], add=True)
            return carry

        lax.fori_loop(0, nch // 2, body, 0)
        plsc.subcore_barrier()

        for z in range(rows_per_tile // ZR):
            r0 = sid * rows_per_tile + z * ZR
            pltpu.sync_copy(acc.at[pl.ds(r0, ZR)], zb_v)
            pltpu.sync_copy(zb_v, out.at[pl.ds(cid * NSEG + r0, ZR)])

    return k


def _make_prep_w(K):
    per_w = EP // NW
    nch = per_w // K

    @functools.partial(
        pl.kernel,
        out_type=(jax.ShapeDtypeStruct((EP,), jnp.int32),
                  jax.ShapeDtypeStruct((EP,), jnp.float32)),
        mesh=_mesh(),
        compiler_params=pltpu.CompilerParams(needs_layout_passes=False),
        scratch_types=[
            pltpu.VMEM((K,), jnp.int32),
            pltpu.VMEM((K,), jnp.int32),
            pltpu.VMEM((K,), jnp.int32),
            pltpu.VMEM((K,), jnp.int32),
            pltpu.VMEM((K,), jnp.int32),
            pltpu.VMEM((K,), jnp.int32),
            pltpu.VMEM((K,), jnp.int32),
            pltpu.VMEM((K,), jnp.float32),
            pltpu.VMEM((NREL * NSEG,), jnp.float32),
            pltpu.SemaphoreType.DMA,
            pltpu.SemaphoreType.DMA,
        ],
    )
    def k(U, esrc, edst, etyp, gout, wout, src_a, src_b, dst_a, dst_b,
          typ_a, typ_b, g_v, w_v, u_v, sem_a, sem_b):
        base = _wid() * per_w
        pltpu.sync_copy(U, u_v)

        def issue(c, s_v, d_v, t_v, sem):
            off = base + c * K
            pltpu.async_copy(esrc.at[pl.ds(off, K)], s_v, sem)
            pltpu.async_copy(edst.at[pl.ds(off, K)], d_v, sem)
            pltpu.async_copy(etyp.at[pl.ds(off, K)], t_v, sem)

        def drain(c, s_v, d_v, t_v, sem):
            off = base + c * K
            pltpu.make_async_copy(esrc.at[pl.ds(off, K)], s_v, sem).wait()
            pltpu.make_async_copy(edst.at[pl.ds(off, K)], d_v, sem).wait()
            pltpu.make_async_copy(etyp.at[pl.ds(off, K)], t_v, sem).wait()

        def compute(c, s_v, d_v, t_v):
            off = base + c * K

            def prep(q, _):
                t = t_v[pl.ds(q * 16, 16)]
                g_v[pl.ds(q * 16, 16)] = t * N_NODES + s_v[pl.ds(q * 16, 16)]
                uix = t * NSEG + d_v[pl.ds(q * 16, 16)]
                w_v[pl.ds(q * 16, 16)] = plsc.load_gather(u_v, [uix])
                return _

            lax.fori_loop(0, K // 16, prep, 0)
            pltpu.sync_copy(g_v, gout.at[pl.ds(off, K)])
            pltpu.sync_copy(w_v, wout.at[pl.ds(off, K)])

        issue(0, src_a, dst_a, typ_a, sem_a)

        def body(i, carry):
            c = 2 * i
            issue(c + 1, src_b, dst_b, typ_b, sem_b)
            drain(c, src_a, dst_a, typ_a, sem_a)
            compute(c, src_a, dst_a, typ_a)

            @pl.when(c + 2 < nch)
            def _():
                issue(c + 2, src_a, dst_a, typ_a, sem_a)

            drain(c + 1, src_b, dst_b, typ_b, sem_b)
            compute(c + 1, src_b, dst_b, typ_b)
            return carry

        lax.fori_loop(0, nch // 2, body, 0)

    return k


def _make_wsegsum(M, K, na, nb):
    assert 16 * (na + nb) * K == M

    @functools.partial(
        pl.kernel,
        out_type=jax.ShapeDtypeStruct((2 * NSEG, EMB), jnp.float32),
        mesh=_mesh(),
        compiler_params=pltpu.CompilerParams(needs_layout_passes=False),
        scratch_types=[
            pltpu.VMEM((K,), jnp.int32),
            pltpu.VMEM((K,), jnp.int32),
            pltpu.VMEM((K,), jnp.int32),
            pltpu.VMEM((K,), jnp.int32),
            pltpu.VMEM((K,), jnp.float32),
            pltpu.VMEM((K,), jnp.float32),
            pltpu.VMEM((K, EMB), jnp.float32),
            pltpu.VMEM((K, EMB), jnp.float32),
            pltpu.VMEM((ZR, EMB), jnp.float32),
            pltpu.VMEM_SHARED((NSEG, EMB), jnp.float32),
            pltpu.SemaphoreType.DMA,
            pltpu.SemaphoreType.DMA,
        ],
    )
    def k(table, gidx, sidx, wts, out,
          gidx_a, gidx_b, sidx_a, sidx_b, w_a, w_b,
          rows_a, rows_b, zb_v, acc, sem_a, sem_b):
        cid = lax.axis_index("c")
        sid = lax.axis_index("s")
        base = jnp.where(cid == 0, sid * (na * K), 16 * na * K + sid * (nb * K))
        nch = jnp.where(cid == 0, na, nb)
        rows_per_tile = NSEG // 16

        _zero_vmem_rows(zb_v, ZR)
        for z in range(rows_per_tile // ZR):
            pltpu.sync_copy(zb_v, acc.at[pl.ds(sid * rows_per_tile + z * ZR, ZR)])
        plsc.subcore_barrier()

        def stage(c, g_v, s_v, w_v):
            off = base + c * K
            pltpu.sync_copy(gidx.at[pl.ds(off, K)], g_v)
            pltpu.sync_copy(sidx.at[pl.ds(off, K)], s_v)
            pltpu.sync_copy(wts.at[pl.ds(off, K)], w_v)

        def scale_scatter(rows_v, w_v, s_v):
            def scale(e, _):
                ws = plsc.load_gather(w_v, [jnp.full((16,), e, jnp.int32)])
                for q in range(EMB // 16):
                    rows_v[e, pl.ds(q * 16, 16)] = (
                        rows_v[e, pl.ds(q * 16, 16)] * ws)
                return _

            lax.fori_loop(0, K, scale, 0)
            pltpu.sync_copy(rows_v, acc.at[s_v], add=True)

        stage(0, gidx_a, sidx_a, w_a)
        pltpu.async_copy(table.at[gidx_a], rows_a, sem_a)

        def body(i, carry):
            c = 2 * i
            stage(c + 1, gidx_b, sidx_b, w_b)
            pltpu.async_copy(table.at[gidx_b], rows_b, sem_b)
            pltpu.make_async_copy(table.at[gidx_a], rows_a, sem_a).wait()
            scale_scatter(rows_a, w_a, sidx_a)

            @pl.when(c + 2 < nch)
            def _():
                stage(c + 2, gidx_a, sidx_a, w_a)
                pltpu.async_copy(table.at[gidx_a], rows_a, sem_a)

            pltpu.make_async_copy(table.at[gidx_b], rows_b, sem_b).wait()
            scale_scatter(rows_b, w_b, sidx_b)
            return carry

        lax.fori_loop(0, nch // 2, body, 0)
        plsc.subcore_barrier()

        for z in range(rows_per_tile // ZR):
            r0 = sid * rows_per_tile + z * ZR
            pltpu.sync_copy(acc.at[pl.ds(r0, ZR)], zb_v)
            pltpu.sync_copy(zb_v, out.at[pl.ds(cid * NSEG + r0, ZR)])

    return k


def _stage1_body(srows, seg, semb, wrow, q1w, q1b, q2w, q2b, h1w,
                 m32, m32t, m32g, out):
    S = srows[...]
    wt = jnp.dot(wrow[...], m32t[...])
    Sw = S * wt.reshape(1, 1, EMB)

    s2 = jnp.sum(Sw * Sw, axis=1)
    n2 = jnp.dot(s2, m32[...])
    inv = 1.0 / jnp.maximum(jnp.sqrt(n2), 1e-12)
    invt = jnp.dot(inv, m32t[...])
    P = Sw * invt[:, None, :]

    segb = seg[...]
    r0 = semb[0, :].reshape(1, 1, EMB)
    r1 = semb[1, :].reshape(1, 1, EMB)
    r2 = semb[2, :].reshape(1, 1, EMB)
    sb = segb[:, :, None]
    G = jnp.where(sb == 0, r0, jnp.where(sb == 1, r1, r2))
    g2 = jnp.sum(G * G, axis=1)
    ng2 = jnp.dot(g2, m32[...])
    ginv = 1.0 / jnp.maximum(jnp.sqrt(ng2), 1e-12)
    ginvt = jnp.dot(ginv, m32t[...])
    P = P + G * ginvt[:, None, :]

    g4 = m32g[...]
    hp = jnp.dot(jnp.dot(P[:, 0, :], g4), q1w[0:4, :])
    for l in range(1, SEQ):
        hp = hp + jnp.dot(jnp.dot(P[:, l, :], g4), q1w[4 * l:4 * l + 4, :])
    h = jax.nn.relu(hp + q1b[...])
    x = jnp.dot(h, q2w[...]) + q2b[...]
    out[...] = jnp.dot(x, h1w[...])


def _scale_body(p0, p1, b, o):
    bb = b[...]
    inv = jnp.where(bb > 0, 1.0 / jnp.maximum(bb, 1e-30), 0.0)
    o[...] = (p0[...] + p1[...]) * inv


def _combine_c_body(p0, p1, d, h1b, rel, rootw, rootb, c, yout, rout, uout):
    r = pl.program_id(1)
    dd = d[...]
    inv = jnp.where(dd > 0, 1.0 / jnp.maximum(dd, 1e-30), 0.0)
    h2x = jax.nn.relu((p0[...] + p1[...]) * inv + h1b[...])
    yout[...] = jnp.dot(h2x, rel[0])[None]

    @pl.when(r == 0)
    def _():
        rout[...] = jnp.dot(h2x, rootw[...]) + rootb[...]
        uout[...] = 1.0 / jnp.maximum(c[...], 1.0)


def _combine_d_body(root, q0, q1, h2w, o):
    x3 = jax.nn.relu(root[...] + q0[...] + q1[...])
    o[...] = jnp.dot(x3, h2w[...])


def _combine_f_body(p0, p1, d, h2b, linw, linb, o):
    dd = d[...]
    inv = jnp.where(dd > 0, 1.0 / jnp.maximum(dd, 1e-30), 0.0)
    h = jax.nn.relu((p0[...] + p1[...]) * inv + h2b[...])
    o[...] = jnp.dot(h, linw[...]) + linb[...]


def _onehot_consts():
    d = np.arange(EMB)
    m32 = np.zeros((EMB, SEQ), np.float32)
    m32[d, d % SEQ] = 1.0
    m32g = np.zeros((EMB, 4), np.float32)
    m32g[d, d // SEQ] = 1.0
    return jnp.asarray(m32), jnp.asarray(m32.T), jnp.asarray(m32g)


def kernel(src, seg, edge_index, hyper_index, edge_type, src_emb, seg_emb, w,
           q1_w, q1_b, q2_w, q2_b, h1_w, h1_b, rg_rel, rg_root, rg_b,
           h2_w, h2_b, lin_w, lin_b):
    m32, m32t, m32g = _onehot_consts()
    f32 = jnp.float32

    srows = _make_gather(src_emb.shape[0], N_NODES * SEQ, 200)(
        src_emb, src.reshape(-1))

    hn = hyper_index[0]
    he = hyper_index[1]
    pad = HP - N_HYPER
    trash_pad = N_NODES + (jnp.arange(pad, dtype=jnp.int32) % (NSEG - N_NODES))
    zero_pad = jnp.zeros((pad,), jnp.int32)
    hn_t = jnp.concatenate([hn, trash_pad])
    he_t = jnp.concatenate([he, trash_pad])
    hn_z = jnp.concatenate([hn, zero_pad])
    he_z = jnp.concatenate([he, zero_pad])
    counts_raw = _make_counts()(hn_t, he_t, edge_index[1], edge_type)
    cnt = pl.pallas_call(
        _counts_reduce_body,
        grid=(16,),
        in_specs=[pl.BlockSpec((NW, 32, EMB), lambda i: (0, i, 0))],
        out_specs=pl.BlockSpec((32, EMB), lambda i: (i, 0)),
        out_shape=jax.ShapeDtypeStruct((CBCAP // EMB, EMB), f32),
    )(counts_raw.reshape(NW, CBCAP // EMB, EMB)).reshape(CBCAP)
    d_col = cnt[:N_NODES].reshape(N_NODES, 1)
    b_col = cnt[NSEG:NSEG + N_NODES].reshape(N_NODES, 1)
    c_rg = cnt[2 * NSEG:6 * NSEG].reshape(1, NREL * NSEG)

    bn = 200
    nb = N_NODES // bn
    full = lambda shape: pl.BlockSpec(shape, lambda i: tuple(0 for _ in shape))
    xl1 = pl.pallas_call(
        _stage1_body,
        grid=(nb,),
        in_specs=[
            pl.BlockSpec((bn, SEQ, EMB), lambda i: (i, 0, 0)),
            pl.BlockSpec((bn, SEQ), lambda i: (i, 0)),
            full((8, EMB)),
            full((1, SEQ)),
            full((EMB, THID)),
            full((1, THID)),
            full((THID, EMB)),
            full((1, EMB)),
            full((EMB, EMB)),
            full((EMB, SEQ)),
            full((SEQ, EMB)),
            full((EMB, 4)),
        ],
        out_specs=pl.BlockSpec((bn, EMB), lambda i: (i, 0)),
        out_shape=jax.ShapeDtypeStruct((N_NODES, EMB), f32),
    )(srows.reshape(N_NODES, SEQ, EMB), seg,
      jnp.concatenate([seg_emb, jnp.zeros((5, EMB), f32)], axis=0),
      w.reshape(1, SEQ), q1_w, q1_b.reshape(1, THID), q2_w,
      q2_b.reshape(1, EMB), h1_w, m32, m32t, m32g)

    seg_hyp = _make_segsum(HP, 160, 54, 10)
    bs = lambda: pl.BlockSpec((1000, EMB), lambda i: (i, 0))
    cs = lambda: pl.BlockSpec((1000, 1), lambda i: (i, 0))
    scale_call = lambda p, col: pl.pallas_call(
        _scale_body,
        grid=(10,),
        in_specs=[bs(), bs(), cs()],
        out_specs=bs(),
        out_shape=jax.ShapeDtypeStruct((N_NODES, EMB), f32),
    )(p[0, :N_NODES], p[1, :N_NODES], col)

    p1h = seg_hyp(xl1, hn_z, he_t).reshape(2, NSEG, EMB)
    e1 = scale_call(p1h, b_col)
    p2h = seg_hyp(e1, he_z, hn_t).reshape(2, NSEG, EMB)

    ub = NREL * NSEG // 10
    yout, root, u = pl.pallas_call(
        _combine_c_body,
        grid=(10, NREL),
        in_specs=[
            pl.BlockSpec((1000, EMB), lambda i, r: (i, 0)),
            pl.BlockSpec((1000, EMB), lambda i, r: (i, 0)),
            pl.BlockSpec((1000, 1), lambda i, r: (i, 0)),
            pl.BlockSpec((1, EMB), lambda i, r: (0, 0)),
            pl.BlockSpec((1, EMB, EMB), lambda i, r: (r, 0, 0)),
            pl.BlockSpec((EMB, EMB), lambda i, r: (0, 0)),
            pl.BlockSpec((1, EMB), lambda i, r: (0, 0)),
            pl.BlockSpec((1, ub), lambda i, r: (0, i)),
        ],
        out_specs=[
            pl.BlockSpec((1, 1000, EMB), lambda i, r: (r, i, 0)),
            pl.BlockSpec((1000, EMB), lambda i, r: (i, 0)),
            pl.BlockSpec((1, ub), lambda i, r: (0, i)),
        ],
        out_shape=[
            jax.ShapeDtypeStruct((NREL, N_NODES, EMB), f32),
            jax.ShapeDtypeStruct((N_NODES, EMB), f32),
            jax.ShapeDtypeStruct((1, NREL * NSEG), f32),
        ],
    )(p2h[0, :N_NODES], p2h[1, :N_NODES], d_col,
      h1_b.reshape(1, EMB), rg_rel, rg_root, rg_b.reshape(1, EMB), c_rg)

    epad = EP - N_EDGES
    ez = jnp.zeros((epad,), jnp.int32)
    esrc_p = jnp.concatenate([edge_index[0], ez])
    edst_p = jnp.concatenate(
        [edge_index[1],
         N_NODES + (jnp.arange(epad, dtype=jnp.int32) % (NSEG - N_NODES))])
    etyp_p = jnp.concatenate([edge_type, ez])
    egidx, ew = _make_prep_w(160)(u.reshape(NREL * NSEG), esrc_p, edst_p, etyp_p)
    qp = _make_wsegsum(EP, 160, 106, 22)(
        yout.reshape(NREL * N_NODES, EMB), egidx, edst_p, ew
    ).reshape(2, NSEG, EMB)

    xl2 = pl.pallas_call(
        _combine_d_body,
        grid=(10,),
        in_specs=[bs(), bs(), bs(), full((EMB, EMB))],
        out_specs=bs(),
        out_shape=jax.ShapeDtypeStruct((N_NODES, EMB), f32),
    )(root, qp[0, :N_NODES], qp[1, :N_NODES], h2_w)

    p3h = seg_hyp(xl2, hn_z, he_t).reshape(2, NSEG, EMB)
    e2 = scale_call(p3h, b_col)
    p4h = seg_hyp(e2, he_z, hn_t).reshape(2, NSEG, EMB)

    out = pl.pallas_call(
        _combine_f_body,
        grid=(10,),
        in_specs=[bs(), bs(), cs(), full((1, EMB)),
                  full((EMB, EMB)), full((1, EMB))],
        out_specs=bs(),
        out_shape=jax.ShapeDtypeStruct((N_NODES, EMB), f32),
    )(p4h[0, :N_NODES], p4h[1, :N_NODES], d_col,
      h2_b.reshape(1, EMB), lin_w, lin_b.reshape(1, EMB))
    return out
